# baseline (speedup 1.0000x reference)
import functools
import jax
import jax.numpy as jnp
from jax import lax
from jax.experimental import pallas as pl
from jax.experimental.pallas import tpu as pltpu
from jax.experimental.pallas import tpu_sc as plsc

_NC = 2
_NS = 16
_NW = _NC * _NS
_L = 16
_CHUNK = 8
_NBUF = 3

_SC_ROWS = 768
_TC_BLK = 256


def _sc_body(rows_per_w, batch, d_model, emb_hbm, pos_hbm, out_hbm, *scratch):
    emb_bufs = scratch[0:_NBUF]
    pos_bufs = scratch[_NBUF:2 * _NBUF]
    emb_sems = scratch[2 * _NBUF:3 * _NBUF]
    pos_sems = scratch[3 * _NBUF:4 * _NBUF]
    out_sems = scratch[4 * _NBUF:5 * _NBUF]

    wid = lax.axis_index("s") * _NC + lax.axis_index("c")
    row0 = wid * rows_per_w
    n_chunks = rows_per_w // _CHUNK
    n_dv = d_model // _L

    def start_in(k):
        b = k % _NBUF
        base = row0 + k * _CHUNK
        h_e = pltpu.async_copy(emb_hbm.at[pl.ds(base, _CHUNK)], emb_bufs[b], emb_sems[b])
        h_p = pltpu.async_copy(pos_hbm.at[pl.ds(base, _CHUNK)], pos_bufs[b], pos_sems[b])
        return h_e, h_p

    def start_out(k):
        b = k % _NBUF
        base = row0 + k * _CHUNK
        return pltpu.async_copy(emb_bufs[b], out_hbm.at[pl.ds(base, _CHUNK)], out_sems[b])

    def compute(k):
        b = k % _NBUF
        emb = emb_bufs[b]
        pos = pos_bufs[b]

        def s_body(s, c):
            @plsc.parallel_loop(0, n_dv // 2, unroll=4)
            def d_body(j):
                d = j * (2 * _L)
                p0 = pos[s, pl.ds(d, _L)]
                p1 = pos[s, pl.ds(d + _L, _L)]
                for bb in range(batch):
                    emb[s, bb, pl.ds(d, _L)] = emb[s, bb, pl.ds(d, _L)] + p0
                    emb[s, bb, pl.ds(d + _L, _L)] = emb[s, bb, pl.ds(d + _L, _L)] + p1

            return c

        lax.fori_loop(0, _CHUNK, s_body, 0)

    in_h = {}
    out_h = {}
    in_h[0] = start_in(0)
    if n_chunks > 1:
        in_h[1] = start_in(1)
    for k in range(n_chunks):
        if k + 2 < n_chunks:
            if k - 1 >= 0:
                out_h[k - 1].wait()
            in_h[k + 2] = start_in(k + 2)
        h_e, h_p = in_h[k]
        h_e.wait()
        h_p.wait()
        compute(k)
        out_h[k] = start_out(k)
    for k in range(max(0, n_chunks - 3), n_chunks):
        out_h[k].wait()


def _sc_part(embedding, pos_table):
    seq_len, batch, d_model = embedding.shape
    rows_per_w = _SC_ROWS // _NW
    mesh = plsc.VectorSubcoreMesh(core_axis_name="c", subcore_axis_name="s")
    body = functools.partial(_sc_body, rows_per_w, batch, d_model)
    scratch = (
        [pltpu.VMEM((_CHUNK, batch, d_model), jnp.float32) for _ in range(_NBUF)]
        + [pltpu.VMEM((_CHUNK, d_model), jnp.float32) for _ in range(_NBUF)]
        + [pltpu.SemaphoreType.DMA for _ in range(3 * _NBUF)]
    )
    return pl.kernel(
        body,
        out_type=jax.ShapeDtypeStruct((_SC_ROWS, batch, d_model), embedding.dtype),
        mesh=mesh,
        scratch_types=scratch,
    )(embedding, pos_table)


def _tc_add_kernel(emb_ref, pos_ref, out_ref):
    out_ref[...] = emb_ref[...] + pos_ref[...][:, None, :]


def _tc_part(embedding, pos_table):
    seq_len, batch, d_model = embedding.shape
    tc_rows = seq_len - _SC_ROWS
    blk0 = _SC_ROWS // _TC_BLK
    return pl.pallas_call(
        _tc_add_kernel,
        grid=(tc_rows // _TC_BLK,),
        in_specs=[
            pl.BlockSpec((_TC_BLK, batch, d_model), lambda i: (i + blk0, 0, 0)),
            pl.BlockSpec((_TC_BLK, d_model), lambda i: (i + blk0, 0)),
        ],
        out_specs=pl.BlockSpec((_TC_BLK, batch, d_model), lambda i: (i, 0, 0)),
        out_shape=jax.ShapeDtypeStruct((tc_rows, batch, d_model), embedding.dtype),
    )(embedding, pos_table)


def kernel(embedding, pos_table):
    sc_out = _sc_part(embedding, pos_table)
    tc_out = _tc_part(embedding, pos_table)
    return jnp.concatenate([sc_out, tc_out], axis=0)


# speedup vs baseline: 1.2810x; 1.2810x over previous
import functools
import jax
import jax.numpy as jnp
from jax import lax
from jax.experimental import pallas as pl
from jax.experimental.pallas import tpu as pltpu
from jax.experimental.pallas import tpu_sc as plsc

_NC = 2
_NS = 16
_NW = _NC * _NS
_L = 16
_CHUNK = 4
_NBUF = 4


def _sc_body(rows_per_w, batch, d_model, emb_hbm, pos_hbm, out_hbm, *scratch):
    emb_bufs = scratch[0:_NBUF]
    pos_bufs = scratch[_NBUF:2 * _NBUF]
    emb_sems = scratch[2 * _NBUF:3 * _NBUF]
    pos_sems = scratch[3 * _NBUF:4 * _NBUF]
    out_sems = scratch[4 * _NBUF:5 * _NBUF]

    wid = lax.axis_index("s") * _NC + lax.axis_index("c")
    row0 = wid * rows_per_w
    n_chunks = rows_per_w // _CHUNK
    n_dv = d_model // _L

    def start_in(k):
        b = k % _NBUF
        base = row0 + k * _CHUNK
        h_e = pltpu.async_copy(emb_hbm.at[pl.ds(base, _CHUNK)], emb_bufs[b], emb_sems[b])
        h_p = pltpu.async_copy(pos_hbm.at[pl.ds(base, _CHUNK)], pos_bufs[b], pos_sems[b])
        return h_e, h_p

    def start_out(k):
        b = k % _NBUF
        base = row0 + k * _CHUNK
        return pltpu.async_copy(emb_bufs[b], out_hbm.at[pl.ds(base, _CHUNK)], out_sems[b])

    def compute(k):
        b = k % _NBUF
        emb = emb_bufs[b]
        pos = pos_bufs[b]

        def s_body(s, c):
            @plsc.parallel_loop(0, n_dv // 2, unroll=4)
            def d_body(j):
                d = j * (2 * _L)
                p0 = pos[s, pl.ds(d, _L)]
                p1 = pos[s, pl.ds(d + _L, _L)]
                for bb in range(batch):
                    emb[s, bb, pl.ds(d, _L)] = emb[s, bb, pl.ds(d, _L)] + p0
                    emb[s, bb, pl.ds(d + _L, _L)] = emb[s, bb, pl.ds(d + _L, _L)] + p1

            return c

        lax.fori_loop(0, _CHUNK, s_body, 0)

    in_h = {}
    out_h = {}
    for k in range(min(_NBUF - 1, n_chunks)):
        in_h[k] = start_in(k)
    for k in range(n_chunks):
        nxt = k + _NBUF - 1
        if nxt < n_chunks:
            prev = nxt - _NBUF
            if prev >= 0:
                out_h[prev].wait()
            in_h[nxt] = start_in(nxt)
        h_e, h_p = in_h.pop(k)
        h_e.wait()
        h_p.wait()
        compute(k)
        out_h[k] = start_out(k)
    for k in range(max(0, n_chunks - _NBUF), n_chunks):
        if k in out_h:
            out_h[k].wait()


def kernel(embedding, pos_table):
    seq_len, batch, d_model = embedding.shape
    rows_per_w = seq_len // _NW
    mesh = plsc.VectorSubcoreMesh(core_axis_name="c", subcore_axis_name="s")
    body = functools.partial(_sc_body, rows_per_w, batch, d_model)
    scratch = (
        [pltpu.VMEM((_CHUNK, batch, d_model), jnp.float32) for _ in range(_NBUF)]
        + [pltpu.VMEM((_CHUNK, d_model), jnp.float32) for _ in range(_NBUF)]
        + [pltpu.SemaphoreType.DMA for _ in range(3 * _NBUF)]
    )
    return pl.kernel(
        body,
        out_type=jax.ShapeDtypeStruct(embedding.shape, embedding.dtype),
        mesh=mesh,
        scratch_types=scratch,
    )(embedding, pos_table)


# speedup vs baseline: 1.3017x; 1.0161x over previous
import functools
import jax
import jax.numpy as jnp
from jax import lax
from jax.experimental import pallas as pl
from jax.experimental.pallas import tpu as pltpu
from jax.experimental.pallas import tpu_sc as plsc

_NC = 2
_NS = 16
_NW = _NC * _NS
_L = 16
_CHUNK = 4
_NBUF = 6


def _sc_body(rows_per_w, batch, d_model, emb_hbm, pos_hbm, out_hbm, *scratch):
    emb_bufs = scratch[0:_NBUF]
    pos_bufs = scratch[_NBUF:2 * _NBUF]
    emb_sems = scratch[2 * _NBUF:3 * _NBUF]
    pos_sems = scratch[3 * _NBUF:4 * _NBUF]
    out_sems = scratch[4 * _NBUF:5 * _NBUF]

    wid = lax.axis_index("s") * _NC + lax.axis_index("c")
    row0 = wid * rows_per_w
    n_chunks = rows_per_w // _CHUNK
    n_dv = d_model // _L

    def start_in(k):
        b = k % _NBUF
        base = row0 + k * _CHUNK
        h_e = pltpu.async_copy(emb_hbm.at[pl.ds(base, _CHUNK)], emb_bufs[b], emb_sems[b])
        h_p = pltpu.async_copy(pos_hbm.at[pl.ds(base, _CHUNK)], pos_bufs[b], pos_sems[b])
        return h_e, h_p

    def start_out(k):
        b = k % _NBUF
        base = row0 + k * _CHUNK
        return pltpu.async_copy(emb_bufs[b], out_hbm.at[pl.ds(base, _CHUNK)], out_sems[b])

    def compute(k):
        b = k % _NBUF
        emb = emb_bufs[b]
        pos = pos_bufs[b]

        def s_body(s, c):
            @plsc.parallel_loop(0, n_dv // 2, unroll=4)
            def d_body(j):
                d = j * (2 * _L)
                p0 = pos[s, pl.ds(d, _L)]
                p1 = pos[s, pl.ds(d + _L, _L)]
                for bb in range(batch):
                    emb[s, bb, pl.ds(d, _L)] = emb[s, bb, pl.ds(d, _L)] + p0
                    emb[s, bb, pl.ds(d + _L, _L)] = emb[s, bb, pl.ds(d + _L, _L)] + p1

            return c

        lax.fori_loop(0, _CHUNK, s_body, 0)

    in_h = {}
    out_h = {}
    for k in range(min(_NBUF - 1, n_chunks)):
        in_h[k] = start_in(k)
    for k in range(n_chunks):
        nxt = k + _NBUF - 1
        if nxt < n_chunks:
            prev = nxt - _NBUF
            if prev >= 0:
                out_h[prev].wait()
            in_h[nxt] = start_in(nxt)
        h_e, h_p = in_h.pop(k)
        h_e.wait()
        h_p.wait()
        compute(k)
        out_h[k] = start_out(k)
    for k in range(max(0, n_chunks - _NBUF), n_chunks):
        if k in out_h:
            out_h[k].wait()


def kernel(embedding, pos_table):
    seq_len, batch, d_model = embedding.shape
    rows_per_w = seq_len // _NW
    mesh = plsc.VectorSubcoreMesh(core_axis_name="c", subcore_axis_name="s")
    body = functools.partial(_sc_body, rows_per_w, batch, d_model)
    scratch = (
        [pltpu.VMEM((_CHUNK, batch, d_model), jnp.float32) for _ in range(_NBUF)]
        + [pltpu.VMEM((_CHUNK, d_model), jnp.float32) for _ in range(_NBUF)]
        + [pltpu.SemaphoreType.DMA for _ in range(3 * _NBUF)]
    )
    return pl.kernel(
        body,
        out_type=jax.ShapeDtypeStruct(embedding.shape, embedding.dtype),
        mesh=mesh,
        scratch_types=scratch,
    )(embedding, pos_table)


# speedup vs baseline: 1.3192x; 1.0134x over previous
import functools
import jax
import jax.numpy as jnp
from jax import lax
from jax.experimental import pallas as pl
from jax.experimental.pallas import tpu as pltpu
from jax.experimental.pallas import tpu_sc as plsc

_NC = 2
_NS = 16
_NW = _NC * _NS
_L = 16
_CHUNK = 4
_NBUF = 6

_SC_ROWS = 1024
_TC_BLK = 256


def _sc_body(row_base, rows_per_w, batch, d_model,
             emb_hbm, pos_hbm, out_hbm, *scratch):
    emb_bufs = scratch[0:_NBUF]
    pos_bufs = scratch[_NBUF:2 * _NBUF]
    emb_sems = scratch[2 * _NBUF:3 * _NBUF]
    pos_sems = scratch[3 * _NBUF:4 * _NBUF]
    out_sems = scratch[4 * _NBUF:5 * _NBUF]

    wid = lax.axis_index("s") * _NC + lax.axis_index("c")
    row0 = row_base + wid * rows_per_w
    n_chunks = rows_per_w // _CHUNK
    n_dv = d_model // _L

    def start_in(k):
        b = k % _NBUF
        base = row0 + k * _CHUNK
        h_e = pltpu.async_copy(emb_hbm.at[pl.ds(base, _CHUNK)], emb_bufs[b], emb_sems[b])
        h_p = pltpu.async_copy(pos_hbm.at[pl.ds(base, _CHUNK)], pos_bufs[b], pos_sems[b])
        return h_e, h_p

    def start_out(k):
        b = k % _NBUF
        base = row0 + k * _CHUNK
        return pltpu.async_copy(emb_bufs[b], out_hbm.at[pl.ds(base, _CHUNK)], out_sems[b])

    def compute(k):
        b = k % _NBUF
        emb = emb_bufs[b]
        pos = pos_bufs[b]

        def s_body(s, c):
            @plsc.parallel_loop(0, n_dv // 2, unroll=4)
            def d_body(j):
                d = j * (2 * _L)
                p0 = pos[s, pl.ds(d, _L)]
                p1 = pos[s, pl.ds(d + _L, _L)]
                for bb in range(batch):
                    emb[s, bb, pl.ds(d, _L)] = emb[s, bb, pl.ds(d, _L)] + p0
                    emb[s, bb, pl.ds(d + _L, _L)] = emb[s, bb, pl.ds(d + _L, _L)] + p1

            return c

        lax.fori_loop(0, _CHUNK, s_body, 0)

    in_h = {}
    out_h = {}
    for k in range(min(_NBUF - 1, n_chunks)):
        in_h[k] = start_in(k)
    for k in range(n_chunks):
        nxt = k + _NBUF - 1
        if nxt < n_chunks:
            prev = nxt - _NBUF
            if prev >= 0:
                out_h[prev].wait()
            in_h[nxt] = start_in(nxt)
        h_e, h_p = in_h.pop(k)
        h_e.wait()
        h_p.wait()
        compute(k)
        out_h[k] = start_out(k)
    for k in range(max(0, n_chunks - _NBUF), n_chunks):
        if k in out_h:
            out_h[k].wait()


def _sc_part(embedding, pos_table):
    seq_len, batch, d_model = embedding.shape
    tc_rows = seq_len - _SC_ROWS
    rows_per_w = _SC_ROWS // _NW
    mesh = plsc.VectorSubcoreMesh(core_axis_name="c", subcore_axis_name="s")
    body = functools.partial(_sc_body, tc_rows, rows_per_w, batch, d_model)
    scratch = (
        [pltpu.VMEM((_CHUNK, batch, d_model), jnp.float32) for _ in range(_NBUF)]
        + [pltpu.VMEM((_CHUNK, d_model), jnp.float32) for _ in range(_NBUF)]
        + [pltpu.SemaphoreType.DMA for _ in range(3 * _NBUF)]
    )
    return pl.kernel(
        body,
        out_type=jax.ShapeDtypeStruct(embedding.shape, embedding.dtype),
        mesh=mesh,
        scratch_types=scratch,
    )(embedding, pos_table)


def _tc_add_kernel(emb_ref, pos_ref, acc_ref, out_ref):
    del acc_ref
    out_ref[...] = emb_ref[...] + pos_ref[...][:, None, :]


def _tc_part(embedding, pos_table, sc_full):
    seq_len, batch, d_model = embedding.shape
    tc_rows = seq_len - _SC_ROWS
    return pl.pallas_call(
        _tc_add_kernel,
        grid=(tc_rows // _TC_BLK,),
        in_specs=[
            pl.BlockSpec((_TC_BLK, batch, d_model), lambda i: (i, 0, 0)),
            pl.BlockSpec((_TC_BLK, d_model), lambda i: (i, 0)),
            pl.BlockSpec(memory_space=pl.ANY),
        ],
        out_specs=pl.BlockSpec((_TC_BLK, batch, d_model), lambda i: (i, 0, 0)),
        out_shape=jax.ShapeDtypeStruct(embedding.shape, embedding.dtype),
        input_output_aliases={2: 0},
    )(embedding, pos_table, sc_full)


def kernel(embedding, pos_table):
    sc_full = _sc_part(embedding, pos_table)
    return _tc_part(embedding, pos_table, sc_full)


# speedup vs baseline: 1.3455x; 1.0200x over previous
import functools
import jax
import jax.numpy as jnp
from jax import lax
from jax.experimental import pallas as pl
from jax.experimental.pallas import tpu as pltpu
from jax.experimental.pallas import tpu_sc as plsc

_NC = 2
_NS = 16
_NW = _NC * _NS
_L = 16
_CHUNK = 4
_NBUF = 6

_SC_ROWS = 512
_TC_BLK = 256


def _sc_body(row_base, rows_per_w, batch, d_model,
             emb_hbm, pos_hbm, out_hbm, *scratch):
    emb_bufs = scratch[0:_NBUF]
    pos_bufs = scratch[_NBUF:2 * _NBUF]
    emb_sems = scratch[2 * _NBUF:3 * _NBUF]
    pos_sems = scratch[3 * _NBUF:4 * _NBUF]
    out_sems = scratch[4 * _NBUF:5 * _NBUF]

    wid = lax.axis_index("s") * _NC + lax.axis_index("c")
    row0 = row_base + wid * rows_per_w
    n_chunks = rows_per_w // _CHUNK
    n_dv = d_model // _L

    def start_in(k):
        b = k % _NBUF
        base = row0 + k * _CHUNK
        h_e = pltpu.async_copy(emb_hbm.at[pl.ds(base, _CHUNK)], emb_bufs[b], emb_sems[b])
        h_p = pltpu.async_copy(pos_hbm.at[pl.ds(base, _CHUNK)], pos_bufs[b], pos_sems[b])
        return h_e, h_p

    def start_out(k):
        b = k % _NBUF
        base = row0 + k * _CHUNK
        return pltpu.async_copy(emb_bufs[b], out_hbm.at[pl.ds(base, _CHUNK)], out_sems[b])

    def compute(k):
        b = k % _NBUF
        emb = emb_bufs[b]
        pos = pos_bufs[b]

        def s_body(s, c):
            @plsc.parallel_loop(0, n_dv // 2, unroll=4)
            def d_body(j):
                d = j * (2 * _L)
                p0 = pos[s, pl.ds(d, _L)]
                p1 = pos[s, pl.ds(d + _L, _L)]
                for bb in range(batch):
                    emb[s, bb, pl.ds(d, _L)] = emb[s, bb, pl.ds(d, _L)] + p0
                    emb[s, bb, pl.ds(d + _L, _L)] = emb[s, bb, pl.ds(d + _L, _L)] + p1

            return c

        lax.fori_loop(0, _CHUNK, s_body, 0)

    in_h = {}
    out_h = {}
    for k in range(min(_NBUF - 1, n_chunks)):
        in_h[k] = start_in(k)
    for k in range(n_chunks):
        nxt = k + _NBUF - 1
        if nxt < n_chunks:
            prev = nxt - _NBUF
            if prev >= 0:
                out_h[prev].wait()
            in_h[nxt] = start_in(nxt)
        h_e, h_p = in_h.pop(k)
        h_e.wait()
        h_p.wait()
        compute(k)
        out_h[k] = start_out(k)
    for k in range(max(0, n_chunks - _NBUF), n_chunks):
        if k in out_h:
            out_h[k].wait()


def _sc_part(embedding, pos_table):
    seq_len, batch, d_model = embedding.shape
    tc_rows = seq_len - _SC_ROWS
    rows_per_w = _SC_ROWS // _NW
    mesh = plsc.VectorSubcoreMesh(core_axis_name="c", subcore_axis_name="s")
    body = functools.partial(_sc_body, tc_rows, rows_per_w, batch, d_model)
    scratch = (
        [pltpu.VMEM((_CHUNK, batch, d_model), jnp.float32) for _ in range(_NBUF)]
        + [pltpu.VMEM((_CHUNK, d_model), jnp.float32) for _ in range(_NBUF)]
        + [pltpu.SemaphoreType.DMA for _ in range(3 * _NBUF)]
    )
    return pl.kernel(
        body,
        out_type=jax.ShapeDtypeStruct(embedding.shape, embedding.dtype),
        mesh=mesh,
        scratch_types=scratch,
    )(embedding, pos_table)


def _tc_add_kernel(emb_ref, pos_ref, acc_ref, out_ref):
    del acc_ref
    out_ref[...] = emb_ref[...] + pos_ref[...][:, None, :]


def _tc_part(embedding, pos_table, sc_full):
    seq_len, batch, d_model = embedding.shape
    tc_rows = seq_len - _SC_ROWS
    return pl.pallas_call(
        _tc_add_kernel,
        grid=(tc_rows // _TC_BLK,),
        in_specs=[
            pl.BlockSpec((_TC_BLK, batch, d_model), lambda i: (i, 0, 0)),
            pl.BlockSpec((_TC_BLK, d_model), lambda i: (i, 0)),
            pl.BlockSpec(memory_space=pl.ANY),
        ],
        out_specs=pl.BlockSpec((_TC_BLK, batch, d_model), lambda i: (i, 0, 0)),
        out_shape=jax.ShapeDtypeStruct(embedding.shape, embedding.dtype),
        input_output_aliases={2: 0},
    )(embedding, pos_table, sc_full)


def kernel(embedding, pos_table):
    sc_full = _sc_part(embedding, pos_table)
    return _tc_part(embedding, pos_table, sc_full)
